# baseline (speedup 1.0000x reference)
import functools

import jax
import jax.numpy as jnp
from jax import lax
from jax.experimental import pallas as pl
from jax.experimental.pallas import tpu as pltpu
from jax.experimental.pallas import tpu_sc as plsc

_PHI = lax.Precision.HIGHEST
_PLO = lax.Precision.DEFAULT

_UNCERT_W = 0.5
_NS_LO = 7
_NS_HI = 2
_CW_JITTER_HALF = 0.05
_NW = 32


def _sc_gather_f(F, conn):
    C = conn.shape[0]
    N = F.shape[1]
    rpw = C // _NW
    mesh = plsc.VectorSubcoreMesh(core_axis_name="c", subcore_axis_name="s")

    @functools.partial(
        pl.kernel,
        mesh=mesh,
        out_type=jax.ShapeDtypeStruct((C, N), jnp.float32),
        scratch_types=[
            pltpu.VMEM((rpw,), jnp.int32),
            pltpu.VMEM((rpw, N), jnp.float32),
            pltpu.SemaphoreType.DMA,
        ],
    )
    def k(F_hbm, conn_hbm, Fc_out, idxc_v, rows_v, sem):
        wid = lax.axis_index("s") * 2 + lax.axis_index("c")
        cbase = wid * rpw
        pltpu.sync_copy(conn_hbm.at[pl.ds(cbase, rpw)], idxc_v)
        pltpu.async_copy(F_hbm.at[idxc_v], rows_v, sem).wait()
        pltpu.sync_copy(rows_v, Fc_out.at[pl.ds(cbase, rpw)])

    return k(F, conn)


def _sc_select(C_u, B, conn, q):
    C = conn.shape[0]
    M = q.shape[0]
    N = C_u.shape[1]
    L = 16
    rpw = C // _NW
    mpw = M // _NW
    nch = C // L
    mesh = plsc.VectorSubcoreMesh(core_axis_name="c", subcore_axis_name="s")

    @functools.partial(
        pl.kernel,
        mesh=mesh,
        compiler_params=pltpu.CompilerParams(needs_layout_passes=False),
        out_type=[
            jax.ShapeDtypeStruct((C, C), jnp.float32),
            jax.ShapeDtypeStruct((M, C), jnp.float32),
        ],
        scratch_types=[
            pltpu.VMEM((rpw,), jnp.int32),
            pltpu.VMEM((mpw,), jnp.int32),
            pltpu.VMEM((C,), jnp.int32),
            pltpu.VMEM((rpw, N), jnp.float32),
            pltpu.VMEM((mpw, N), jnp.float32),
            pltpu.VMEM((rpw, C), jnp.float32),
            pltpu.VMEM((mpw, C), jnp.float32),
            pltpu.SemaphoreType.DMA,
            pltpu.SemaphoreType.DMA,
        ],
    )
    def k(Cu_hbm, B_hbm, conn_hbm, q_hbm, Cucc_out, Bqc_out,
          idxc_v, idxq_v, conn_v, rows_v, rowsb_v, selc_v, selb_v, sem, semb):
        wid = lax.axis_index("s") * 2 + lax.axis_index("c")
        cbase = wid * rpw
        mbase = wid * mpw
        pltpu.sync_copy(conn_hbm.at[pl.ds(cbase, rpw)], idxc_v)
        pltpu.sync_copy(q_hbm.at[pl.ds(mbase, mpw)], idxq_v)
        pltpu.sync_copy(conn_hbm, conn_v)
        b_cp = pltpu.async_copy(B_hbm.at[idxq_v], rowsb_v, semb)
        cu_cp = pltpu.async_copy(Cu_hbm.at[idxc_v], rows_v, sem)

        def select_rows(nrows, src_v, dst_v):
            for j in range(nch):
                cols = conn_v[pl.ds(j * L, L)]

                def row(r, _):
                    ridx = jnp.full((L,), r, jnp.int32)
                    vals = plsc.load_gather(src_v, [ridx, cols])
                    dst_v[r, pl.ds(j * L, L)] = vals
                    return 0
                lax.fori_loop(0, nrows, row, 0)

        b_cp.wait()
        select_rows(mpw, rowsb_v, selb_v)
        pltpu.sync_copy(selb_v, Bqc_out.at[pl.ds(mbase, mpw)])
        cu_cp.wait()
        select_rows(rpw, rows_v, selc_v)
        pltpu.sync_copy(selc_v, Cucc_out.at[pl.ds(cbase, rpw)])

    return k(C_u, B, conn, q)


def _tc_sigma(F_c, C_x, s0):
    C, N = F_c.shape
    BK = 512
    nk = N // BK

    def body(fc_ref, fck_ref, cx_ref, s0_ref, sig_ref, sc_ref):
        kk = pl.program_id(0)

        @pl.when(kk == 0)
        def _():
            sig_ref[...] = jnp.zeros_like(sig_ref)
            sc_ref[...] = jnp.dot(fc_ref[...], s0_ref[...],
                                  preferred_element_type=jnp.float32,
                                  precision=_PLO)

        g = jnp.dot(fc_ref[...].astype(jnp.bfloat16), cx_ref[...],
                    preferred_element_type=jnp.float32, precision=_PLO)
        sig_ref[...] += lax.dot_general(
            g, fck_ref[...], (((1,), (1,)), ((), ())),
            preferred_element_type=jnp.float32, precision=_PLO)

    return pl.pallas_call(
        body,
        grid=(nk,),
        in_specs=[
            pl.BlockSpec((C, N), lambda k: (0, 0)),
            pl.BlockSpec((C, BK), lambda k: (0, k)),
            pl.BlockSpec((N, BK), lambda k: (0, k)),
            pl.BlockSpec((N, 1), lambda k: (0, 0)),
        ],
        out_specs=[
            pl.BlockSpec((C, C), lambda k: (0, 0)),
            pl.BlockSpec((C, 1), lambda k: (0, 0)),
        ],
        out_shape=[
            jax.ShapeDtypeStruct((C, C), jnp.float32),
            jax.ShapeDtypeStruct((C, 1), jnp.float32),
        ],
    )(F_c, F_c, C_x, s0)


def _tc_update(conn2d, Cu_cc, Bq_c_in, C_w, Sig_fc, s_c, poly, y2d, n_out):
    C = Cu_cc.shape[0]
    M = Bq_c_in.shape[0]
    N = n_out

    def body(conn_ref, cucc_ref, bqc_ref, cw_ref, sig_ref, sc_ref, poly_ref,
             y_ref, out_ref):
        conn_b = conn_ref[...]
        rows = lax.broadcasted_iota(jnp.int32, (N, C), 0)
        E = (rows == conn_b).astype(jnp.float32)

        Bq_c = bqc_ref[...]
        Sig = sig_ref[...] + cucc_ref[...]

        sc0 = sc_ref[...]
        sc = jnp.where(sc0 > 0, sc0, _UNCERT_W)
        Lq = jnp.dot(Bq_c, sc, preferred_element_type=jnp.float32,
                     precision=_PLO)

        c0, c1, c2, c3 = (poly_ref[0], poly_ref[1], poly_ref[2], poly_ref[3])
        dp = c1 + 2.0 * c2 * Lq + 3.0 * c3 * Lq * Lq
        pred = c0 + Lq * (c1 + Lq * (c2 + Lq * c3))
        H = dp * Bq_c

        T1 = jnp.dot(H, Sig, preferred_element_type=jnp.float32,
                     precision=_PLO)
        S = lax.dot_general(T1, H, (((1,), (1,)), ((), ())),
                            preferred_element_type=jnp.float32,
                            precision=_PLO) + cw_ref[...]

        r = y_ref[...] - pred

        u0 = jnp.max(jnp.sum(jnp.abs(S), axis=1))
        l0 = jnp.float32(_CW_JITTER_HALF)
        ri = lax.broadcasted_iota(jnp.int32, (M, M), 0)
        ci = lax.broadcasted_iota(jnp.int32, (M, M), 1)
        eye = (ri == ci).astype(jnp.float32)

        def ns_scaled(_, carry):
            X, l, u = carry
            t = 2.0 / (l + u)
            Xs = t * X
            SX = jnp.dot(S, Xs, preferred_element_type=jnp.float32,
                         precision=_PLO)
            Xn = 2.0 * Xs - jnp.dot(Xs, SX,
                                    preferred_element_type=jnp.float32,
                                    precision=_PLO)
            tl = t * l
            tu = t * u
            ln = jnp.minimum(tl * (2.0 - tl), tu * (2.0 - tu))
            return Xn, ln, jnp.float32(1.0)

        def ns_polish(_, X):
            SX = jnp.dot(S, X, preferred_element_type=jnp.float32,
                         precision=_PHI)
            return 2.0 * X - jnp.dot(X, SX,
                                     preferred_element_type=jnp.float32,
                                     precision=_PHI)

        X, _, _ = lax.fori_loop(0, _NS_LO, ns_scaled, (eye, l0, u0))
        X = lax.fori_loop(0, _NS_HI, ns_polish, X)
        K1 = lax.dot_general(Sig, H, (((1,), (1,)), ((), ())),
                             preferred_element_type=jnp.float32,
                             precision=_PLO)
        K2 = jnp.dot(K1, X, preferred_element_type=jnp.float32,
                     precision=_PLO)
        kvec = jnp.dot(K2, r, preferred_element_type=jnp.float32,
                       precision=_PLO)
        out_c = jnp.maximum(sc + kvec, 0.0)

        hi = out_c.astype(jnp.bfloat16).astype(jnp.float32)
        lo = out_c - hi
        out_ref[...] = (
            jnp.dot(E, hi, preferred_element_type=jnp.float32, precision=_PLO)
            + jnp.dot(E, lo, preferred_element_type=jnp.float32,
                      precision=_PLO))

    return pl.pallas_call(
        body,
        in_specs=[
            pl.BlockSpec((1, C), lambda: (0, 0)),
            pl.BlockSpec((C, C), lambda: (0, 0)),
            pl.BlockSpec((M, C), lambda: (0, 0)),
            pl.BlockSpec((M, M), lambda: (0, 0)),
            pl.BlockSpec((C, C), lambda: (0, 0)),
            pl.BlockSpec((C, 1), lambda: (0, 0)),
            pl.BlockSpec(memory_space=pltpu.SMEM),
            pl.BlockSpec((M, 1), lambda: (0, 0)),
        ],
        out_specs=pl.BlockSpec((n_out, 1), lambda: (0, 0)),
        out_shape=jax.ShapeDtypeStruct((n_out, 1), jnp.float32),
    )(conn2d, Cu_cc, Bq_c_in, C_w, Sig_fc, s_c, poly, y2d)


def kernel(F, B, C_u, C_w, C_x, StateInit, poly_c, q, y, connections):
    N = F.shape[0]
    C = connections.shape[0]
    M = y.shape[0]
    F_c = _sc_gather_f(F, connections)
    Cu_cc, Bq_c = _sc_select(C_u, B, connections, q)
    Sig_fc, s_c = _tc_sigma(F_c, C_x.astype(jnp.bfloat16),
                            StateInit.reshape(N, 1))
    out2d = _tc_update(connections.reshape(1, C), Cu_cc, Bq_c, C_w, Sig_fc,
                       s_c, poly_c, y.reshape(M, 1), N)
    return out2d.reshape(N)


# speedup vs baseline: 1.0599x; 1.0599x over previous
import functools

import jax
import jax.numpy as jnp
from jax import lax
from jax.experimental import pallas as pl
from jax.experimental.pallas import tpu as pltpu
from jax.experimental.pallas import tpu_sc as plsc

_PHI = lax.Precision.HIGHEST
_PLO = lax.Precision.DEFAULT

_UNCERT_W = 0.5
_NS_LO = 7
_NS_HI = 2
_CW_JITTER_HALF = 0.05
_NW = 32


def _sc_gather_f(F, conn):
    C = conn.shape[0]
    N = F.shape[1]
    rpw = C // _NW
    mesh = plsc.VectorSubcoreMesh(core_axis_name="c", subcore_axis_name="s")

    @functools.partial(
        pl.kernel,
        mesh=mesh,
        out_type=jax.ShapeDtypeStruct((C, N), jnp.float32),
        scratch_types=[
            pltpu.VMEM((rpw,), jnp.int32),
            pltpu.VMEM((rpw, N), jnp.float32),
            pltpu.SemaphoreType.DMA,
        ],
    )
    def k(F_hbm, conn_hbm, Fc_out, idxc_v, rows_v, sem):
        wid = lax.axis_index("s") * 2 + lax.axis_index("c")
        cbase = wid * rpw
        pltpu.sync_copy(conn_hbm.at[pl.ds(cbase, rpw)], idxc_v)
        pltpu.async_copy(F_hbm.at[idxc_v], rows_v, sem).wait()
        pltpu.sync_copy(rows_v, Fc_out.at[pl.ds(cbase, rpw)])

    return k(F, conn)


def _sc_select(C_u, B, conn, q):
    C = conn.shape[0]
    M = q.shape[0]
    N = C_u.shape[1]
    L = 16
    rpw = C // _NW
    mpw = M // _NW
    nch = C // L
    mesh = plsc.VectorSubcoreMesh(core_axis_name="c", subcore_axis_name="s")

    @functools.partial(
        pl.kernel,
        mesh=mesh,
        compiler_params=pltpu.CompilerParams(needs_layout_passes=False),
        out_type=[
            jax.ShapeDtypeStruct((C, C), jnp.float32),
            jax.ShapeDtypeStruct((M, C), jnp.float32),
        ],
        scratch_types=[
            pltpu.VMEM((rpw,), jnp.int32),
            pltpu.VMEM((mpw,), jnp.int32),
            pltpu.VMEM((C,), jnp.int32),
            pltpu.VMEM((rpw, N), jnp.float32),
            pltpu.VMEM((mpw, N), jnp.float32),
            pltpu.VMEM((rpw, C), jnp.float32),
            pltpu.VMEM((mpw, C), jnp.float32),
            pltpu.SemaphoreType.DMA,
            pltpu.SemaphoreType.DMA,
        ],
    )
    def k(Cu_hbm, B_hbm, conn_hbm, q_hbm, Cucc_out, Bqc_out,
          idxc_v, idxq_v, conn_v, rows_v, rowsb_v, selc_v, selb_v, sem, semb):
        wid = lax.axis_index("s") * 2 + lax.axis_index("c")
        cbase = wid * rpw
        mbase = wid * mpw
        pltpu.sync_copy(conn_hbm.at[pl.ds(cbase, rpw)], idxc_v)
        pltpu.sync_copy(q_hbm.at[pl.ds(mbase, mpw)], idxq_v)
        pltpu.sync_copy(conn_hbm, conn_v)
        b_cp = pltpu.async_copy(B_hbm.at[idxq_v], rowsb_v, semb)
        cu_cp = pltpu.async_copy(Cu_hbm.at[idxc_v], rows_v, sem)

        def select_rows(nrows, src_v, dst_v):
            for j in range(nch):
                cols = conn_v[pl.ds(j * L, L)]

                def row(r, _):
                    ridx = jnp.full((L,), r, jnp.int32)
                    vals = plsc.load_gather(src_v, [ridx, cols])
                    dst_v[r, pl.ds(j * L, L)] = vals
                    return 0
                lax.fori_loop(0, nrows, row, 0)

        b_cp.wait()
        select_rows(mpw, rowsb_v, selb_v)
        pltpu.sync_copy(selb_v, Bqc_out.at[pl.ds(mbase, mpw)])
        cu_cp.wait()
        select_rows(rpw, rows_v, selc_v)
        pltpu.sync_copy(selc_v, Cucc_out.at[pl.ds(cbase, rpw)])

    return k(C_u, B, conn, q)


def _tc_sigma(F_c, C_x, s0):
    C, N = F_c.shape
    BK = 512
    nk = N // BK

    def body(fc_ref, fck_ref, cx_ref, s0_ref, sig_ref, sc_ref):
        kk = pl.program_id(0)

        @pl.when(kk == 0)
        def _():
            sig_ref[...] = jnp.zeros_like(sig_ref)
            sc_ref[...] = jnp.dot(fc_ref[...], s0_ref[...],
                                  preferred_element_type=jnp.float32,
                                  precision=_PLO)

        g = jnp.dot(fc_ref[...], cx_ref[...],
                    preferred_element_type=jnp.float32, precision=_PLO)
        sig_ref[...] += lax.dot_general(
            g, fck_ref[...], (((1,), (1,)), ((), ())),
            preferred_element_type=jnp.float32, precision=_PLO)

    return pl.pallas_call(
        body,
        grid=(nk,),
        in_specs=[
            pl.BlockSpec((C, N), lambda k: (0, 0)),
            pl.BlockSpec((C, BK), lambda k: (0, k)),
            pl.BlockSpec((N, BK), lambda k: (0, k)),
            pl.BlockSpec((N, 1), lambda k: (0, 0)),
        ],
        out_specs=[
            pl.BlockSpec((C, C), lambda k: (0, 0)),
            pl.BlockSpec((C, 1), lambda k: (0, 0)),
        ],
        out_shape=[
            jax.ShapeDtypeStruct((C, C), jnp.float32),
            jax.ShapeDtypeStruct((C, 1), jnp.float32),
        ],
    )(F_c, F_c, C_x, s0)


def _tc_update(conn2d, Cu_cc, Bq_c_in, C_w, Sig_fc, s_c, poly, y2d, n_out):
    C = Cu_cc.shape[0]
    M = Bq_c_in.shape[0]
    N = n_out

    def body(conn_ref, cucc_ref, bqc_ref, cw_ref, sig_ref, sc_ref, poly_ref,
             y_ref, out_ref):
        conn_b = conn_ref[...]
        rows = lax.broadcasted_iota(jnp.int32, (N, C), 0)
        E = (rows == conn_b).astype(jnp.float32)

        Bq_c = bqc_ref[...]
        Sig = sig_ref[...] + cucc_ref[...]

        sc0 = sc_ref[...]
        sc = jnp.where(sc0 > 0, sc0, _UNCERT_W)
        Lq = jnp.dot(Bq_c, sc, preferred_element_type=jnp.float32,
                     precision=_PLO)

        c0, c1, c2, c3 = (poly_ref[0], poly_ref[1], poly_ref[2], poly_ref[3])
        dp = c1 + 2.0 * c2 * Lq + 3.0 * c3 * Lq * Lq
        pred = c0 + Lq * (c1 + Lq * (c2 + Lq * c3))
        H = dp * Bq_c

        T1 = jnp.dot(H, Sig, preferred_element_type=jnp.float32,
                     precision=_PLO)
        S = lax.dot_general(T1, H, (((1,), (1,)), ((), ())),
                            preferred_element_type=jnp.float32,
                            precision=_PLO) + cw_ref[...]

        r = y_ref[...] - pred

        u0 = jnp.max(jnp.sum(jnp.abs(S), axis=1))
        l0 = jnp.float32(_CW_JITTER_HALF)
        ri = lax.broadcasted_iota(jnp.int32, (M, M), 0)
        ci = lax.broadcasted_iota(jnp.int32, (M, M), 1)
        eye = (ri == ci).astype(jnp.float32)

        def ns_scaled(_, carry):
            X, l, u = carry
            t = 2.0 / (l + u)
            Xs = t * X
            SX = jnp.dot(S, Xs, preferred_element_type=jnp.float32,
                         precision=_PLO)
            Xn = 2.0 * Xs - jnp.dot(Xs, SX,
                                    preferred_element_type=jnp.float32,
                                    precision=_PLO)
            tl = t * l
            tu = t * u
            ln = jnp.minimum(tl * (2.0 - tl), tu * (2.0 - tu))
            return Xn, ln, jnp.float32(1.0)

        def ns_polish(_, X):
            SX = jnp.dot(S, X, preferred_element_type=jnp.float32,
                         precision=_PHI)
            return 2.0 * X - jnp.dot(X, SX,
                                     preferred_element_type=jnp.float32,
                                     precision=_PHI)

        X, _, _ = lax.fori_loop(0, _NS_LO, ns_scaled, (eye, l0, u0))
        X = lax.fori_loop(0, _NS_HI, ns_polish, X)
        K1 = lax.dot_general(Sig, H, (((1,), (1,)), ((), ())),
                             preferred_element_type=jnp.float32,
                             precision=_PLO)
        K2 = jnp.dot(K1, X, preferred_element_type=jnp.float32,
                     precision=_PLO)
        kvec = jnp.dot(K2, r, preferred_element_type=jnp.float32,
                       precision=_PLO)
        out_c = jnp.maximum(sc + kvec, 0.0)

        hi = out_c.astype(jnp.bfloat16).astype(jnp.float32)
        lo = out_c - hi
        out_ref[...] = (
            jnp.dot(E, hi, preferred_element_type=jnp.float32, precision=_PLO)
            + jnp.dot(E, lo, preferred_element_type=jnp.float32,
                      precision=_PLO))

    return pl.pallas_call(
        body,
        in_specs=[
            pl.BlockSpec((1, C), lambda: (0, 0)),
            pl.BlockSpec((C, C), lambda: (0, 0)),
            pl.BlockSpec((M, C), lambda: (0, 0)),
            pl.BlockSpec((M, M), lambda: (0, 0)),
            pl.BlockSpec((C, C), lambda: (0, 0)),
            pl.BlockSpec((C, 1), lambda: (0, 0)),
            pl.BlockSpec(memory_space=pltpu.SMEM),
            pl.BlockSpec((M, 1), lambda: (0, 0)),
        ],
        out_specs=pl.BlockSpec((n_out, 1), lambda: (0, 0)),
        out_shape=jax.ShapeDtypeStruct((n_out, 1), jnp.float32),
    )(conn2d, Cu_cc, Bq_c_in, C_w, Sig_fc, s_c, poly, y2d)


def kernel(F, B, C_u, C_w, C_x, StateInit, poly_c, q, y, connections):
    N = F.shape[0]
    C = connections.shape[0]
    M = y.shape[0]
    F_c = _sc_gather_f(F, connections)
    Cu_cc, Bq_c = _sc_select(C_u, B, connections, q)
    Sig_fc, s_c = _tc_sigma(F_c, C_x, StateInit.reshape(N, 1))
    out2d = _tc_update(connections.reshape(1, C), Cu_cc, Bq_c, C_w, Sig_fc,
                       s_c, poly_c, y.reshape(M, 1), N)
    return out2d.reshape(N)
